# baseline (speedup 1.0000x reference)
import functools

import jax
import jax.numpy as jnp
from jax import lax
from jax.experimental import pallas as pl
from jax.experimental.pallas import tpu as pltpu
from jax.experimental.pallas import tpu_sc as plsc

EPS = 1e-15
NC = 2
NS = 16
NW = NC * NS
LANES = 16


def _sc_dots(z, srcs, dsts, *, chunk, interpret=False):
    n, d = z.shape
    (e_total,) = srcs.shape
    assert e_total % NW == 0
    e_per_w = e_total // NW
    assert e_per_w % (2 * chunk) == 0 and chunk % LANES == 0 and chunk % 8 == 0
    assert chunk <= 128
    n_pairs = e_per_w // (2 * chunk)
    n_groups = chunk // LANES

    mesh = plsc.VectorSubcoreMesh(core_axis_name="c", subcore_axis_name="s",
                                  num_cores=NC, num_subcores=NS)

    @functools.partial(
        pl.kernel,
        out_type=jax.ShapeDtypeStruct((e_total,), jnp.float32),
        mesh=mesh,
        interpret=interpret,
        compiler_params=pltpu.CompilerParams(
            use_tc_tiling_on_sc=False, needs_layout_passes=False),
        scratch_types=[
            pltpu.VMEM((e_per_w,), jnp.int32),
            pltpu.VMEM((e_per_w,), jnp.int32),
            pltpu.VMEM((2, chunk, d), jnp.float32),
            pltpu.VMEM((2, chunk, d), jnp.float32),
            pltpu.VMEM((2, chunk), jnp.float32),
            pltpu.SemaphoreType.DMA((2,)),
            pltpu.SemaphoreType.DMA((2,)),
        ],
    )
    def k(z_hbm, src_hbm, dst_hbm, out_hbm, src_idx, dst_idx, src_rows,
          dst_rows, dots, gsem, ssem):
        wid = lax.axis_index("s") * NC + lax.axis_index("c")
        wbase = wid * e_per_w
        lanes = lax.iota(jnp.int32, LANES)

        pltpu.sync_copy(src_hbm.at[pl.ds(wbase, e_per_w)], src_idx)
        pltpu.sync_copy(dst_hbm.at[pl.ds(wbase, e_per_w)], dst_idx)

        def issue(g, slot):
            sl = pl.ds(g * chunk, chunk)
            pltpu.async_copy(z_hbm.at[src_idx.at[sl]], src_rows.at[slot],
                             gsem.at[slot])
            pltpu.async_copy(z_hbm.at[dst_idx.at[sl]], dst_rows.at[slot],
                             gsem.at[slot])

        def wait_gathers(slot):
            pltpu.make_async_copy(z_hbm.at[pl.ds(0, chunk)],
                                  src_rows.at[slot], gsem.at[slot]).wait()
            pltpu.make_async_copy(z_hbm.at[pl.ds(0, chunk)],
                                  dst_rows.at[slot], gsem.at[slot]).wait()

        def drain_scatter(slot):
            pltpu.make_async_copy(out_hbm.at[pl.ds(0, chunk)],
                                  dots.at[slot], ssem.at[slot]).wait()

        def compute(g, slot):
            sref = src_rows.at[slot]
            dref = dst_rows.at[slot]

            last = jnp.full((LANES,), LANES - 1, jnp.int32)

            @plsc.parallel_loop(0, n_groups)
            def _group(g16):
                e0 = g16 * LANES
                accs = [None] * LANES
                for c in range(d // LANES):
                    sl = pl.ds(c * LANES, LANES)
                    for kk in range(LANES):
                        prod = sref[e0 + kk, sl] * dref[e0 + kk, sl]
                        accs[kk] = prod if c == 0 else accs[kk] + prod
                out_vec = jnp.zeros((LANES,), jnp.float32)
                for kk in range(LANES):
                    tot = jnp.take_along_axis(
                        plsc.cumsum(accs[kk]), last, axis=0)
                    out_vec = jnp.where(lanes == kk, tot, out_vec)
                dots[slot, pl.ds(e0, LANES)] = out_vec

            pltpu.async_copy(dots.at[slot],
                             out_hbm.at[pl.ds(wbase + g * chunk, chunk)],
                             ssem.at[slot])

        issue(0, 0)
        issue(1, 1)

        def pair(i, _):
            a = 2 * i
            wait_gathers(0)

            @pl.when(i > 0)
            def _():
                drain_scatter(0)

            compute(a, 0)

            @pl.when(i < n_pairs - 1)
            def _():
                issue(a + 2, 0)

            wait_gathers(1)

            @pl.when(i > 0)
            def _():
                drain_scatter(1)

            compute(a + 1, 1)

            @pl.when(i < n_pairs - 1)
            def _():
                issue(a + 3, 1)

            return 0

        lax.fori_loop(0, n_pairs, pair, 0)
        drain_scatter(0)
        drain_scatter(1)

    return k(z, srcs, dsts)


def _tc_loss(dots, e_pos, *, interpret=False):
    (e_total,) = dots.shape
    assert e_total == 2 * e_pos and e_pos % 128 == 0
    rows = e_pos // 128
    d3 = dots.reshape(2, rows, 128)
    inv = 1.0 / e_pos

    def body(d_ref, out_ref):
        p_pos = jax.nn.sigmoid(d_ref[0])
        p_neg = jax.nn.sigmoid(d_ref[1])
        q = jnp.maximum(1.0 - p_neg, 0.0)
        t = -jnp.log(p_pos + EPS) - jnp.log(q + EPS)
        out_ref[0, 0] = jnp.sum(t) * inv

    out = pl.pallas_call(
        body,
        out_specs=pl.BlockSpec(memory_space=pltpu.SMEM),
        out_shape=jax.ShapeDtypeStruct((1, 1), jnp.float32),
        interpret=interpret,
    )(d3)
    return out[0, 0]


def kernel(z, pos_edge_index, neg_edge_index, *, interpret=False):
    e_pos = pos_edge_index.shape[1]
    srcs = jnp.concatenate([pos_edge_index[0], neg_edge_index[0]])
    dsts = jnp.concatenate([pos_edge_index[1], neg_edge_index[1]])
    e_total = srcs.shape[0]
    e_per_w = e_total // NW
    chunk = 80 if e_per_w % 160 == 0 else e_per_w
    dots = _sc_dots(z, srcs, dsts, chunk=chunk, interpret=interpret)
    return _tc_loss(dots, e_pos, interpret=interpret)


# speedup vs baseline: 2.2191x; 2.2191x over previous
import functools

import jax
import jax.numpy as jnp
from jax import lax
from jax.experimental import pallas as pl
from jax.experimental.pallas import tpu as pltpu
from jax.experimental.pallas import tpu_sc as plsc

EPS = 1e-15
NC = 2
NS = 16
NW = NC * NS
LANES = 16


def _sc_dots(z, srcs, dsts, *, chunk, interpret=False):
    n, d = z.shape
    (e_total,) = srcs.shape
    assert e_total % NW == 0
    e_per_w = e_total // NW
    assert e_per_w % (2 * chunk) == 0 and chunk % LANES == 0 and chunk % 8 == 0
    assert chunk <= 128
    n_pairs = e_per_w // (2 * chunk)
    n_groups = chunk // LANES

    mesh = plsc.VectorSubcoreMesh(core_axis_name="c", subcore_axis_name="s",
                                  num_cores=NC, num_subcores=NS)

    @functools.partial(
        pl.kernel,
        out_type=jax.ShapeDtypeStruct((e_total,), jnp.float32),
        mesh=mesh,
        interpret=interpret,
        compiler_params=pltpu.CompilerParams(
            use_tc_tiling_on_sc=False, needs_layout_passes=False),
        scratch_types=[
            pltpu.VMEM((e_per_w,), jnp.int32),
            pltpu.VMEM((e_per_w,), jnp.int32),
            pltpu.VMEM((2, chunk, d), jnp.float32),
            pltpu.VMEM((2, chunk, d), jnp.float32),
            pltpu.VMEM((2, chunk), jnp.float32),
            pltpu.SemaphoreType.DMA((2,)),
            pltpu.SemaphoreType.DMA((2,)),
        ],
    )
    def k(z_hbm, src_hbm, dst_hbm, out_hbm, src_idx, dst_idx, src_rows,
          dst_rows, dots, gsem, ssem):
        wid = lax.axis_index("s") * NC + lax.axis_index("c")
        wbase = wid * e_per_w
        lanes = lax.iota(jnp.int32, LANES)

        pltpu.sync_copy(src_hbm.at[pl.ds(wbase, e_per_w)], src_idx)
        pltpu.sync_copy(dst_hbm.at[pl.ds(wbase, e_per_w)], dst_idx)

        def issue(g, slot):
            sl = pl.ds(g * chunk, chunk)
            pltpu.async_copy(z_hbm.at[src_idx.at[sl]], src_rows.at[slot],
                             gsem.at[slot])
            pltpu.async_copy(z_hbm.at[dst_idx.at[sl]], dst_rows.at[slot],
                             gsem.at[slot])

        def wait_gathers(slot):
            pltpu.make_async_copy(z_hbm.at[pl.ds(0, chunk)],
                                  src_rows.at[slot], gsem.at[slot]).wait()
            pltpu.make_async_copy(z_hbm.at[pl.ds(0, chunk)],
                                  dst_rows.at[slot], gsem.at[slot]).wait()

        def drain_scatter(slot):
            pltpu.make_async_copy(out_hbm.at[pl.ds(0, chunk)],
                                  dots.at[slot], ssem.at[slot]).wait()

        def compute(g, slot):
            sref = src_rows.at[slot]
            dref = dst_rows.at[slot]

            rots = [((lanes + sh) % LANES).astype(jnp.int32)
                    for sh in (8, 4, 2, 1)]
            lane0 = lanes == 0

            @plsc.parallel_loop(0, chunk, unroll=4)
            def _edge(e):
                prods = [
                    sref[e, pl.ds(c * LANES, LANES)]
                    * dref[e, pl.ds(c * LANES, LANES)]
                    for c in range(d // LANES)
                ]
                while len(prods) > 1:
                    prods = [prods[i] + prods[i + 1]
                             for i in range(0, len(prods), 2)]
                t = prods[0]
                for r in rots:
                    t = t + jnp.take_along_axis(t, r, axis=0)
                idx = jnp.full((LANES,), e, jnp.int32)
                plsc.store_scatter(dots.at[slot], [idx], t, mask=lane0)

            pltpu.async_copy(dots.at[slot],
                             out_hbm.at[pl.ds(wbase + g * chunk, chunk)],
                             ssem.at[slot])

        issue(0, 0)
        issue(1, 1)

        def pair(i, _):
            a = 2 * i
            wait_gathers(0)

            @pl.when(i > 0)
            def _():
                drain_scatter(0)

            compute(a, 0)

            @pl.when(i < n_pairs - 1)
            def _():
                issue(a + 2, 0)

            wait_gathers(1)

            @pl.when(i > 0)
            def _():
                drain_scatter(1)

            compute(a + 1, 1)

            @pl.when(i < n_pairs - 1)
            def _():
                issue(a + 3, 1)

            return 0

        lax.fori_loop(0, n_pairs, pair, 0)
        drain_scatter(0)
        drain_scatter(1)

    return k(z, srcs, dsts)


def _tc_loss(dots, e_pos, *, interpret=False):
    (e_total,) = dots.shape
    assert e_total == 2 * e_pos and e_pos % 128 == 0
    rows = e_pos // 128
    d3 = dots.reshape(2, rows, 128)
    inv = 1.0 / e_pos

    def body(d_ref, out_ref):
        p_pos = jax.nn.sigmoid(d_ref[0])
        p_neg = jax.nn.sigmoid(d_ref[1])
        q = jnp.maximum(1.0 - p_neg, 0.0)
        t = -jnp.log(p_pos + EPS) - jnp.log(q + EPS)
        out_ref[0, 0] = jnp.sum(t) * inv

    out = pl.pallas_call(
        body,
        out_specs=pl.BlockSpec(memory_space=pltpu.SMEM),
        out_shape=jax.ShapeDtypeStruct((1, 1), jnp.float32),
        interpret=interpret,
    )(d3)
    return out[0, 0]


def kernel(z, pos_edge_index, neg_edge_index, *, interpret=False):
    e_pos = pos_edge_index.shape[1]
    srcs = jnp.concatenate([pos_edge_index[0], neg_edge_index[0]])
    dsts = jnp.concatenate([pos_edge_index[1], neg_edge_index[1]])
    e_total = srcs.shape[0]
    e_per_w = e_total // NW
    chunk = 80 if e_per_w % 160 == 0 else e_per_w
    dots = _sc_dots(z, srcs, dsts, chunk=chunk, interpret=interpret)
    return _tc_loss(dots, e_pos, interpret=interpret)


# speedup vs baseline: 2.7862x; 1.2556x over previous
import functools

import jax
import jax.numpy as jnp
from jax import lax
from jax.experimental import pallas as pl
from jax.experimental.pallas import tpu as pltpu
from jax.experimental.pallas import tpu_sc as plsc

EPS = 1e-15
NC = 2
NS = 16
NW = NC * NS
LANES = 16


def _sc_dots(z, srcs, dsts, *, chunk, interpret=False):
    n, d = z.shape
    (e_total,) = srcs.shape
    assert e_total % NW == 0
    e_per_w = e_total // NW
    assert e_per_w % (2 * chunk) == 0 and chunk % LANES == 0 and chunk % 8 == 0
    assert chunk <= 128
    n_pairs = e_per_w // (2 * chunk)
    n_groups = chunk // LANES

    mesh = plsc.VectorSubcoreMesh(core_axis_name="c", subcore_axis_name="s",
                                  num_cores=NC, num_subcores=NS)

    @functools.partial(
        pl.kernel,
        out_type=jax.ShapeDtypeStruct((e_total,), jnp.float32),
        mesh=mesh,
        interpret=interpret,
        compiler_params=pltpu.CompilerParams(
            use_tc_tiling_on_sc=False, needs_layout_passes=False),
        scratch_types=[
            pltpu.VMEM((e_per_w,), jnp.int32),
            pltpu.VMEM((e_per_w,), jnp.int32),
            pltpu.VMEM((2, chunk, d), jnp.bfloat16),
            pltpu.VMEM((2, chunk, d), jnp.bfloat16),
            pltpu.VMEM((2, chunk), jnp.float32),
            pltpu.SemaphoreType.DMA((2,)),
            pltpu.SemaphoreType.DMA((2,)),
        ],
    )
    def k(z_hbm, src_hbm, dst_hbm, out_hbm, src_idx, dst_idx, src_rows,
          dst_rows, dots, gsem, ssem):
        wid = lax.axis_index("s") * NC + lax.axis_index("c")
        wbase = wid * e_per_w
        lanes = lax.iota(jnp.int32, LANES)

        pltpu.sync_copy(src_hbm.at[pl.ds(wbase, e_per_w)], src_idx)
        pltpu.sync_copy(dst_hbm.at[pl.ds(wbase, e_per_w)], dst_idx)

        def issue(g, slot):
            sl = pl.ds(g * chunk, chunk)
            pltpu.async_copy(z_hbm.at[src_idx.at[sl]], src_rows.at[slot],
                             gsem.at[slot])
            pltpu.async_copy(z_hbm.at[dst_idx.at[sl]], dst_rows.at[slot],
                             gsem.at[slot])

        def wait_gathers(slot):
            pltpu.make_async_copy(z_hbm.at[pl.ds(0, chunk)],
                                  src_rows.at[slot], gsem.at[slot]).wait()
            pltpu.make_async_copy(z_hbm.at[pl.ds(0, chunk)],
                                  dst_rows.at[slot], gsem.at[slot]).wait()

        def drain_scatter(slot):
            pltpu.make_async_copy(out_hbm.at[pl.ds(0, chunk)],
                                  dots.at[slot], ssem.at[slot]).wait()

        def compute(g, slot):
            sref = src_rows.at[slot]
            dref = dst_rows.at[slot]

            rots = [((lanes + sh) % LANES).astype(jnp.int32)
                    for sh in (8, 4, 2, 1)]
            lane0 = lanes == 0

            @plsc.parallel_loop(0, chunk, unroll=4)
            def _edge(e):
                prods = []
                for c in range(d // (2 * LANES)):
                    sl = pl.ds(c * 2 * LANES, 2 * LANES)
                    sa, sb = plsc.unpack(sref[e, sl],
                                         format=plsc.PackFormat.INTERLEAVED)
                    da, db = plsc.unpack(dref[e, sl],
                                         format=plsc.PackFormat.INTERLEAVED)
                    prods.append(sa * da)
                    prods.append(sb * db)
                while len(prods) > 1:
                    prods = [prods[i] + prods[i + 1]
                             for i in range(0, len(prods), 2)]
                t = prods[0]
                for r in rots:
                    t = t + jnp.take_along_axis(t, r, axis=0)
                idx = jnp.full((LANES,), e, jnp.int32)
                plsc.store_scatter(dots.at[slot], [idx], t, mask=lane0)

            pltpu.async_copy(dots.at[slot],
                             out_hbm.at[pl.ds(wbase + g * chunk, chunk)],
                             ssem.at[slot])

        issue(0, 0)
        issue(1, 1)

        def pair(i, _):
            a = 2 * i
            wait_gathers(0)

            @pl.when(i > 0)
            def _():
                drain_scatter(0)

            compute(a, 0)

            @pl.when(i < n_pairs - 1)
            def _():
                issue(a + 2, 0)

            wait_gathers(1)

            @pl.when(i > 0)
            def _():
                drain_scatter(1)

            compute(a + 1, 1)

            @pl.when(i < n_pairs - 1)
            def _():
                issue(a + 3, 1)

            return 0

        lax.fori_loop(0, n_pairs, pair, 0)
        drain_scatter(0)
        drain_scatter(1)

    return k(z, srcs, dsts)


def _tc_loss(dots, e_pos, *, interpret=False):
    (e_total,) = dots.shape
    assert e_total == 2 * e_pos and e_pos % 128 == 0
    rows = e_pos // 128
    d3 = dots.reshape(2, rows, 128)
    inv = 1.0 / e_pos

    def body(d_ref, out_ref):
        p_pos = jax.nn.sigmoid(d_ref[0])
        p_neg = jax.nn.sigmoid(d_ref[1])
        q = jnp.maximum(1.0 - p_neg, 0.0)
        t = -jnp.log(p_pos + EPS) - jnp.log(q + EPS)
        out_ref[0, 0] = jnp.sum(t) * inv

    out = pl.pallas_call(
        body,
        out_specs=pl.BlockSpec(memory_space=pltpu.SMEM),
        out_shape=jax.ShapeDtypeStruct((1, 1), jnp.float32),
        interpret=interpret,
    )(d3)
    return out[0, 0]


def kernel(z, pos_edge_index, neg_edge_index, *, interpret=False):
    e_pos = pos_edge_index.shape[1]
    srcs = jnp.concatenate([pos_edge_index[0], neg_edge_index[0]])
    dsts = jnp.concatenate([pos_edge_index[1], neg_edge_index[1]])
    e_total = srcs.shape[0]
    e_per_w = e_total // NW
    chunk = 80 if e_per_w % 160 == 0 else e_per_w
    dots = _sc_dots(z.astype(jnp.bfloat16), srcs, dsts, chunk=chunk,
                    interpret=interpret)
    return _tc_loss(dots, e_pos, interpret=interpret)


# speedup vs baseline: 2.9954x; 1.0751x over previous
import functools

import jax
import jax.numpy as jnp
from jax import lax
from jax.experimental import pallas as pl
from jax.experimental.pallas import tpu as pltpu
from jax.experimental.pallas import tpu_sc as plsc

EPS = 1e-15
NC = 2
NS = 16
NW = NC * NS
LANES = 16


def _sc_dots(z, srcs, dsts, *, chunk, interpret=False):
    n, d = z.shape
    (e_total,) = srcs.shape
    assert e_total % NW == 0
    e_per_w = e_total // NW
    assert e_per_w % (2 * chunk) == 0 and chunk % LANES == 0 and chunk % 8 == 0
    assert chunk <= 128
    n_pairs = e_per_w // (2 * chunk)
    n_groups = chunk // LANES

    mesh = plsc.VectorSubcoreMesh(core_axis_name="c", subcore_axis_name="s",
                                  num_cores=NC, num_subcores=NS)

    @functools.partial(
        pl.kernel,
        out_type=jax.ShapeDtypeStruct((e_total,), jnp.float32),
        mesh=mesh,
        interpret=interpret,
        compiler_params=pltpu.CompilerParams(
            use_tc_tiling_on_sc=False, needs_layout_passes=False),
        scratch_types=[
            pltpu.VMEM((e_per_w,), jnp.int32),
            pltpu.VMEM((e_per_w,), jnp.int32),
            pltpu.VMEM((2, chunk, d), jnp.bfloat16),
            pltpu.VMEM((2, chunk, d), jnp.bfloat16),
            pltpu.VMEM((2, chunk), jnp.float32),
            pltpu.SemaphoreType.DMA((2,)),
            pltpu.SemaphoreType.DMA((2,)),
        ],
    )
    def k(z_hbm, src_hbm, dst_hbm, out_hbm, src_idx, dst_idx, src_rows,
          dst_rows, dots, gsem, ssem):
        wid = lax.axis_index("s") * NC + lax.axis_index("c")
        wbase = wid * e_per_w
        lanes = lax.iota(jnp.int32, LANES)

        pltpu.sync_copy(src_hbm.at[pl.ds(wbase, e_per_w)], src_idx)
        pltpu.sync_copy(dst_hbm.at[pl.ds(wbase, e_per_w)], dst_idx)

        def issue(g, slot):
            sl = pl.ds(g * chunk, chunk)
            pltpu.async_copy(z_hbm.at[src_idx.at[sl]], src_rows.at[slot],
                             gsem.at[slot])
            pltpu.async_copy(z_hbm.at[dst_idx.at[sl]], dst_rows.at[slot],
                             gsem.at[slot])

        def wait_gathers(slot):
            pltpu.make_async_copy(z_hbm.at[pl.ds(0, chunk)],
                                  src_rows.at[slot], gsem.at[slot]).wait()
            pltpu.make_async_copy(z_hbm.at[pl.ds(0, chunk)],
                                  dst_rows.at[slot], gsem.at[slot]).wait()

        def drain_scatter(slot):
            pltpu.make_async_copy(out_hbm.at[pl.ds(0, chunk)],
                                  dots.at[slot], ssem.at[slot]).wait()

        def compute(g, slot):
            sref = src_rows.at[slot]
            dref = dst_rows.at[slot]

            rots = [((lanes + sh) % LANES).astype(jnp.int32)
                    for sh in (8, 4, 2, 1)]
            lane0 = lanes == 0

            @plsc.parallel_loop(0, chunk, unroll=4)
            def _edge(e):
                prods = []
                for c in range(d // (2 * LANES)):
                    sl = pl.ds(c * 2 * LANES, 2 * LANES)
                    pr = sref[e, sl] * dref[e, sl]
                    pa, pb = plsc.unpack(pr,
                                         format=plsc.PackFormat.INTERLEAVED)
                    prods.append(pa)
                    prods.append(pb)
                while len(prods) > 1:
                    prods = [prods[i] + prods[i + 1]
                             for i in range(0, len(prods), 2)]
                t = prods[0]
                for r in rots:
                    t = t + jnp.take_along_axis(t, r, axis=0)
                idx = jnp.full((LANES,), e, jnp.int32)
                plsc.store_scatter(dots.at[slot], [idx], t, mask=lane0)

            pltpu.async_copy(dots.at[slot],
                             out_hbm.at[pl.ds(wbase + g * chunk, chunk)],
                             ssem.at[slot])

        issue(0, 0)
        issue(1, 1)

        def pair(i, _):
            a = 2 * i
            wait_gathers(0)

            @pl.when(i > 0)
            def _():
                drain_scatter(0)

            compute(a, 0)

            @pl.when(i < n_pairs - 1)
            def _():
                issue(a + 2, 0)

            wait_gathers(1)

            @pl.when(i > 0)
            def _():
                drain_scatter(1)

            compute(a + 1, 1)

            @pl.when(i < n_pairs - 1)
            def _():
                issue(a + 3, 1)

            return 0

        lax.fori_loop(0, n_pairs, pair, 0)
        drain_scatter(0)
        drain_scatter(1)

    return k(z, srcs, dsts)


def _tc_loss(dots, e_pos, *, interpret=False):
    (e_total,) = dots.shape
    assert e_total == 2 * e_pos and e_pos % 128 == 0
    rows = e_pos // 128
    d3 = dots.reshape(2, rows, 128)
    inv = 1.0 / e_pos

    def body(d_ref, out_ref):
        p_pos = jax.nn.sigmoid(d_ref[0])
        p_neg = jax.nn.sigmoid(d_ref[1])
        q = jnp.maximum(1.0 - p_neg, 0.0)
        t = -jnp.log(p_pos + EPS) - jnp.log(q + EPS)
        out_ref[0, 0] = jnp.sum(t) * inv

    out = pl.pallas_call(
        body,
        out_specs=pl.BlockSpec(memory_space=pltpu.SMEM),
        out_shape=jax.ShapeDtypeStruct((1, 1), jnp.float32),
        interpret=interpret,
    )(d3)
    return out[0, 0]


def kernel(z, pos_edge_index, neg_edge_index, *, interpret=False):
    e_pos = pos_edge_index.shape[1]
    srcs = jnp.concatenate([pos_edge_index[0], neg_edge_index[0]])
    dsts = jnp.concatenate([pos_edge_index[1], neg_edge_index[1]])
    e_total = srcs.shape[0]
    e_per_w = e_total // NW
    chunk = 80 if e_per_w % 160 == 0 else e_per_w
    dots = _sc_dots(z.astype(jnp.bfloat16), srcs, dsts, chunk=chunk,
                    interpret=interpret)
    return _tc_loss(dots, e_pos, interpret=interpret)
